# baseline (speedup 1.0000x reference)
import jax
import jax.numpy as jnp
from jax import lax
from jax.experimental import pallas as pl
from jax.experimental.pallas import tpu as pltpu

B, K, L, H = 64, 64, 64, 64

_DNT = (((1,), (1,)), ((), ()))

_BF = jnp.bfloat16
_F32 = jnp.float32


def _mmT(x, w):
    return lax.dot_general(x, w, _DNT, preferred_element_type=_F32)


def _mmT16(x, w):
    return lax.dot_general(x.astype(_BF), w.astype(_BF), _DNT,
                           preferred_element_type=_F32)


def _sum_leading(x):
    while x.shape[0] > 1:
        h = x.shape[0] // 2
        x = x[:h] + x[h:]
    return x[0]


def _sum_axis1(x):
    while x.shape[1] > 1:
        h = x.shape[1] // 2
        x = x[:, :h] + x[:, h:]
    return x[:, 0]


def _fused_kernel(h_ue_ref, h_ap_ref, e_ref,
                  wa1_ref, ba1_ref, wa2_ref, ba2_ref,
                  wu1_ref, bu1_ref, wu2_ref, bu2_ref,
                  wih_ue_ref, bih_ue_ref, whh_ue_ref, bhh_ue_ref,
                  wih_ap_ref, bih_ap_ref, whh_ap_ref, bhh_ap_ref,
                  we1_ref, be1_ref, we2_ref, be2_ref,
                  h_ue_out_ref, h_ap_out_ref, e_out_ref):
    b = pl.program_id(0)
    hu = h_ue_ref[b]
    ha = h_ap_ref[b]
    e16 = e_ref[0].astype(_BF)

    a_src = _mmT(ha, wa1_ref[:, :H]) + ba1_ref[...]
    t = _mmT16(e16, wa1_ref[:, H:])
    t = jax.nn.relu(t.reshape(L, K, H) + a_src[:, None, :])
    s = _sum_leading(t)
    m_ue = _mmT(s, wa2_ref[...]) * (1.0 / L) + ba2_ref[...]

    u_src = _mmT(hu, wu1_ref[:, :H]) + bu1_ref[...]
    t = _mmT16(e16, wu1_ref[:, H:])
    t = jax.nn.relu(t.reshape(L, K, H) + u_src[None, :, :])
    s = _sum_axis1(t)
    m_ap = _mmT(s, wu2_ref[...]) * (1.0 / K) + bu2_ref[...]

    def gru(x, h, wih_ref, bih_ref, whh_ref, bhh_ref):
        gi = _mmT(x, wih_ref[...]) + bih_ref[...]
        gh = _mmT(h, whh_ref[...]) + bhh_ref[...]
        r = jax.nn.sigmoid(gi[:, :H] + gh[:, :H])
        z = jax.nn.sigmoid(gi[:, H:2 * H] + gh[:, H:2 * H])
        n = jnp.tanh(gi[:, 2 * H:] + r * gh[:, 2 * H:])
        return (1.0 - z) * n + z * h

    hu_new = gru(m_ue, hu, wih_ue_ref, bih_ue_ref, whh_ue_ref, bhh_ue_ref)
    ha_new = gru(m_ap, ha, wih_ap_ref, bih_ap_ref, whh_ap_ref, bhh_ap_ref)
    h_ue_out_ref[b] = hu_new
    h_ap_out_ref[b] = ha_new

    s_u = _mmT(hu_new, we1_ref[:, :H])
    s_a = _mmT(ha_new, we1_ref[:, H:2 * H]) + be1_ref[...]
    t = _mmT16(e16, we1_ref[:, 2 * H:])
    t = jax.nn.relu(t.reshape(L, K, H) + s_a[:, None, :]
                    + s_u[None, :, :]).reshape(L * K, H)
    e_out_ref[0] = _mmT16(t, we2_ref[...]) + be2_ref[...]


def kernel(h_ue, h_ap, e, W_a2u_1, b_a2u_1, W_a2u_2, b_a2u_2,
           W_u2a_1, b_u2a_1, W_u2a_2, b_u2a_2,
           Wih_ue, bih_ue, Whh_ue, bhh_ue, Wih_ap, bih_ap, Whh_ap, bhh_ap,
           W_e_1, b_e_1, W_e_2, b_e_2):
    batch3 = lambda s: pl.BlockSpec((1,) + s, lambda b: (b, 0, 0))
    fixed = lambda s: pl.BlockSpec(s, lambda b: (0,) * len(s))

    out_shapes = (
        jax.ShapeDtypeStruct((B, K, H), _F32),
        jax.ShapeDtypeStruct((B, L, H), _F32),
        jax.ShapeDtypeStruct((B, L * K, H), _F32),
    )

    in_specs = [
        fixed((B, K, H)), fixed((B, L, H)), batch3((L * K, H)),
        fixed((H, 2 * H)), fixed((1, H)), fixed((H, H)), fixed((1, H)),
        fixed((H, 2 * H)), fixed((1, H)), fixed((H, H)), fixed((1, H)),
        fixed((3 * H, H)), fixed((1, 3 * H)), fixed((3 * H, H)), fixed((1, 3 * H)),
        fixed((3 * H, H)), fixed((1, 3 * H)), fixed((3 * H, H)), fixed((1, 3 * H)),
        fixed((H, 3 * H)), fixed((1, H)), fixed((H, H)), fixed((1, H)),
    ]

    return pl.pallas_call(
        _fused_kernel,
        grid=(B,),
        in_specs=in_specs,
        out_specs=[fixed((B, K, H)), fixed((B, L, H)), batch3((L * K, H))],
        out_shape=out_shapes,
        compiler_params=pltpu.CompilerParams(
            dimension_semantics=("arbitrary",),
        ),
    )(h_ue, h_ap, e,
      W_a2u_1, b_a2u_1.reshape(1, H), W_a2u_2, b_a2u_2.reshape(1, H),
      W_u2a_1, b_u2a_1.reshape(1, H), W_u2a_2, b_u2a_2.reshape(1, H),
      Wih_ue, bih_ue.reshape(1, 3 * H), Whh_ue, bhh_ue.reshape(1, 3 * H),
      Wih_ap, bih_ap.reshape(1, 3 * H), Whh_ap, bhh_ap.reshape(1, 3 * H),
      W_e_1, b_e_1.reshape(1, H), W_e_2, b_e_2.reshape(1, H))


# speedup vs baseline: 1.0944x; 1.0944x over previous
import jax
import jax.numpy as jnp
from jax import lax
from jax.experimental import pallas as pl
from jax.experimental.pallas import tpu as pltpu

B, K, L, H = 64, 64, 64, 64
C = 2

_DNT = (((1,), (1,)), ((), ()))

_BF = jnp.bfloat16
_F32 = jnp.float32


def _mmT(x, w):
    return lax.dot_general(x, w, _DNT, preferred_element_type=_F32)


def _mmT16(x, w):
    return lax.dot_general(x.astype(_BF), w.astype(_BF), _DNT,
                           preferred_element_type=_F32)


def _sum_axis(x, axis):
    while x.shape[axis] > 1:
        h = x.shape[axis] // 2
        lo = lax.slice_in_dim(x, 0, h, axis=axis)
        hi = lax.slice_in_dim(x, h, 2 * h, axis=axis)
        x = lo + hi
    return lax.squeeze(x, (axis,))


def _fused_kernel(h_ue_ref, h_ap_ref, e_ref,
                  wa1_ref, ba1_ref, wa2_ref, ba2_ref,
                  wu1_ref, bu1_ref, wu2_ref, bu2_ref,
                  wih_ue_ref, bih_ue_ref, whh_ue_ref, bhh_ue_ref,
                  wih_ap_ref, bih_ap_ref, whh_ap_ref, bhh_ap_ref,
                  we1_ref, be1_ref, we2_ref, be2_ref,
                  h_ue_out_ref, h_ap_out_ref, e_out_ref):
    b0 = pl.program_id(0) * C
    hu = h_ue_ref[pl.ds(b0, C)]
    ha = h_ap_ref[pl.ds(b0, C)]
    e16 = e_ref[...].reshape(C * L * K, H).astype(_BF)

    a_src = _mmT(ha.reshape(C * L, H), wa1_ref[:, :H]) + ba1_ref[...]
    t = _mmT16(e16, wa1_ref[:, H:])
    t = jax.nn.relu(t.reshape(C, L, K, H) + a_src.reshape(C, L, 1, H))
    s = _sum_axis(t, 1)
    m_ue = (_mmT(s.reshape(C * K, H), wa2_ref[...]) * (1.0 / L)
            + ba2_ref[...])

    u_src = _mmT(hu.reshape(C * K, H), wu1_ref[:, :H]) + bu1_ref[...]
    t = _mmT16(e16, wu1_ref[:, H:])
    t = jax.nn.relu(t.reshape(C, L, K, H) + u_src.reshape(C, 1, K, H))
    s = _sum_axis(t, 2)
    m_ap = (_mmT(s.reshape(C * L, H), wu2_ref[...]) * (1.0 / K)
            + bu2_ref[...])

    def gru(x, h, wih_ref, bih_ref, whh_ref, bhh_ref):
        gi = _mmT(x, wih_ref[...]) + bih_ref[...]
        gh = _mmT(h, whh_ref[...]) + bhh_ref[...]
        r = jax.nn.sigmoid(gi[:, :H] + gh[:, :H])
        z = jax.nn.sigmoid(gi[:, H:2 * H] + gh[:, H:2 * H])
        n = jnp.tanh(gi[:, 2 * H:] + r * gh[:, 2 * H:])
        return (1.0 - z) * n + z * h

    hu_new = gru(m_ue, hu.reshape(C * K, H),
                 wih_ue_ref, bih_ue_ref, whh_ue_ref, bhh_ue_ref)
    ha_new = gru(m_ap, ha.reshape(C * L, H),
                 wih_ap_ref, bih_ap_ref, whh_ap_ref, bhh_ap_ref)
    h_ue_out_ref[pl.ds(b0, C)] = hu_new.reshape(C, K, H)
    h_ap_out_ref[pl.ds(b0, C)] = ha_new.reshape(C, L, H)

    s_u = _mmT(hu_new, we1_ref[:, :H])
    s_a = _mmT(ha_new, we1_ref[:, H:2 * H]) + be1_ref[...]
    t = _mmT16(e16, we1_ref[:, 2 * H:])
    t = jax.nn.relu(t.reshape(C, L, K, H) + s_a.reshape(C, L, 1, H)
                    + s_u.reshape(C, 1, K, H))
    e_out_ref[...] = (_mmT16(t.reshape(C * L * K, H), we2_ref[...])
                      + be2_ref[...]).reshape(C, L * K, H)


def kernel(h_ue, h_ap, e, W_a2u_1, b_a2u_1, W_a2u_2, b_a2u_2,
           W_u2a_1, b_u2a_1, W_u2a_2, b_u2a_2,
           Wih_ue, bih_ue, Whh_ue, bhh_ue, Wih_ap, bih_ap, Whh_ap, bhh_ap,
           W_e_1, b_e_1, W_e_2, b_e_2):
    chunk3 = lambda s: pl.BlockSpec((C,) + s, lambda b: (b, 0, 0))
    fixed = lambda s: pl.BlockSpec(s, lambda b: (0,) * len(s))

    out_shapes = (
        jax.ShapeDtypeStruct((B, K, H), _F32),
        jax.ShapeDtypeStruct((B, L, H), _F32),
        jax.ShapeDtypeStruct((B, L * K, H), _F32),
    )

    in_specs = [
        fixed((B, K, H)), fixed((B, L, H)), chunk3((L * K, H)),
        fixed((H, 2 * H)), fixed((1, H)), fixed((H, H)), fixed((1, H)),
        fixed((H, 2 * H)), fixed((1, H)), fixed((H, H)), fixed((1, H)),
        fixed((3 * H, H)), fixed((1, 3 * H)), fixed((3 * H, H)), fixed((1, 3 * H)),
        fixed((3 * H, H)), fixed((1, 3 * H)), fixed((3 * H, H)), fixed((1, 3 * H)),
        fixed((H, 3 * H)), fixed((1, H)), fixed((H, H)), fixed((1, H)),
    ]

    return pl.pallas_call(
        _fused_kernel,
        grid=(B // C,),
        in_specs=in_specs,
        out_specs=[fixed((B, K, H)), fixed((B, L, H)), chunk3((L * K, H))],
        out_shape=out_shapes,
        compiler_params=pltpu.CompilerParams(
            dimension_semantics=("arbitrary",),
        ),
    )(h_ue, h_ap, e,
      W_a2u_1, b_a2u_1.reshape(1, H), W_a2u_2, b_a2u_2.reshape(1, H),
      W_u2a_1, b_u2a_1.reshape(1, H), W_u2a_2, b_u2a_2.reshape(1, H),
      Wih_ue, bih_ue.reshape(1, 3 * H), Whh_ue, bhh_ue.reshape(1, 3 * H),
      Wih_ap, bih_ap.reshape(1, 3 * H), Whh_ap, bhh_ap.reshape(1, 3 * H),
      W_e_1, b_e_1.reshape(1, H), W_e_2, b_e_2.reshape(1, H))


# speedup vs baseline: 1.1407x; 1.0423x over previous
import jax
import jax.numpy as jnp
from jax import lax
from jax.experimental import pallas as pl
from jax.experimental.pallas import tpu as pltpu

B, K, L, H = 64, 64, 64, 64
C = 4

_DNT = (((1,), (1,)), ((), ()))

_BF = jnp.bfloat16
_F32 = jnp.float32


def _mmT(x, w):
    return lax.dot_general(x, w, _DNT, preferred_element_type=_F32)


def _mmT16(x, w):
    return lax.dot_general(x.astype(_BF), w.astype(_BF), _DNT,
                           preferred_element_type=_F32)


def _sum_axis(x, axis):
    while x.shape[axis] > 1:
        h = x.shape[axis] // 2
        lo = lax.slice_in_dim(x, 0, h, axis=axis)
        hi = lax.slice_in_dim(x, h, 2 * h, axis=axis)
        x = lo + hi
    return lax.squeeze(x, (axis,))


def _fused_kernel(h_ue_ref, h_ap_ref, e_ref,
                  wa1_ref, ba1_ref, wa2_ref, ba2_ref,
                  wu1_ref, bu1_ref, wu2_ref, bu2_ref,
                  wih_ue_ref, bih_ue_ref, whh_ue_ref, bhh_ue_ref,
                  wih_ap_ref, bih_ap_ref, whh_ap_ref, bhh_ap_ref,
                  we1_ref, be1_ref, we2_ref, be2_ref,
                  h_ue_out_ref, h_ap_out_ref, e_out_ref):
    b0 = pl.program_id(0) * C
    hu = h_ue_ref[pl.ds(b0, C)]
    ha = h_ap_ref[pl.ds(b0, C)]
    e16 = e_ref[...].reshape(C * L * K, H).astype(_BF)

    a_src = _mmT(ha.reshape(C * L, H), wa1_ref[:, :H]) + ba1_ref[...]
    t = _mmT16(e16, wa1_ref[:, H:])
    t = jax.nn.relu(t.reshape(C, L, K, H) + a_src.reshape(C, L, 1, H))
    s = _sum_axis(t, 1)
    m_ue = (_mmT(s.reshape(C * K, H), wa2_ref[...]) * (1.0 / L)
            + ba2_ref[...])

    u_src = _mmT(hu.reshape(C * K, H), wu1_ref[:, :H]) + bu1_ref[...]
    t = _mmT16(e16, wu1_ref[:, H:])
    t = jax.nn.relu(t.reshape(C, L, K, H) + u_src.reshape(C, 1, K, H))
    s = _sum_axis(t, 2)
    m_ap = (_mmT(s.reshape(C * L, H), wu2_ref[...]) * (1.0 / K)
            + bu2_ref[...])

    def gru(x, h, wih_ref, bih_ref, whh_ref, bhh_ref):
        gi = _mmT(x, wih_ref[...]) + bih_ref[...]
        gh = _mmT(h, whh_ref[...]) + bhh_ref[...]
        r = jax.nn.sigmoid(gi[:, :H] + gh[:, :H])
        z = jax.nn.sigmoid(gi[:, H:2 * H] + gh[:, H:2 * H])
        n = jnp.tanh(gi[:, 2 * H:] + r * gh[:, 2 * H:])
        return (1.0 - z) * n + z * h

    hu_new = gru(m_ue, hu.reshape(C * K, H),
                 wih_ue_ref, bih_ue_ref, whh_ue_ref, bhh_ue_ref)
    ha_new = gru(m_ap, ha.reshape(C * L, H),
                 wih_ap_ref, bih_ap_ref, whh_ap_ref, bhh_ap_ref)
    h_ue_out_ref[pl.ds(b0, C)] = hu_new.reshape(C, K, H)
    h_ap_out_ref[pl.ds(b0, C)] = ha_new.reshape(C, L, H)

    s_u = _mmT(hu_new, we1_ref[:, :H])
    s_a = _mmT(ha_new, we1_ref[:, H:2 * H]) + be1_ref[...]
    t = _mmT16(e16, we1_ref[:, 2 * H:])
    t = jax.nn.relu(t.reshape(C, L, K, H) + s_a.reshape(C, L, 1, H)
                    + s_u.reshape(C, 1, K, H))
    e_out_ref[...] = (_mmT16(t.reshape(C * L * K, H), we2_ref[...])
                      + be2_ref[...]).reshape(C, L * K, H)


def kernel(h_ue, h_ap, e, W_a2u_1, b_a2u_1, W_a2u_2, b_a2u_2,
           W_u2a_1, b_u2a_1, W_u2a_2, b_u2a_2,
           Wih_ue, bih_ue, Whh_ue, bhh_ue, Wih_ap, bih_ap, Whh_ap, bhh_ap,
           W_e_1, b_e_1, W_e_2, b_e_2):
    chunk3 = lambda s: pl.BlockSpec((C,) + s, lambda b: (b, 0, 0))
    fixed = lambda s: pl.BlockSpec(s, lambda b: (0,) * len(s))

    out_shapes = (
        jax.ShapeDtypeStruct((B, K, H), _F32),
        jax.ShapeDtypeStruct((B, L, H), _F32),
        jax.ShapeDtypeStruct((B, L * K, H), _F32),
    )

    in_specs = [
        fixed((B, K, H)), fixed((B, L, H)), chunk3((L * K, H)),
        fixed((H, 2 * H)), fixed((1, H)), fixed((H, H)), fixed((1, H)),
        fixed((H, 2 * H)), fixed((1, H)), fixed((H, H)), fixed((1, H)),
        fixed((3 * H, H)), fixed((1, 3 * H)), fixed((3 * H, H)), fixed((1, 3 * H)),
        fixed((3 * H, H)), fixed((1, 3 * H)), fixed((3 * H, H)), fixed((1, 3 * H)),
        fixed((H, 3 * H)), fixed((1, H)), fixed((H, H)), fixed((1, H)),
    ]

    return pl.pallas_call(
        _fused_kernel,
        grid=(B // C,),
        in_specs=in_specs,
        out_specs=[fixed((B, K, H)), fixed((B, L, H)), chunk3((L * K, H))],
        out_shape=out_shapes,
        compiler_params=pltpu.CompilerParams(
            dimension_semantics=("arbitrary",),
        ),
    )(h_ue, h_ap, e,
      W_a2u_1, b_a2u_1.reshape(1, H), W_a2u_2, b_a2u_2.reshape(1, H),
      W_u2a_1, b_u2a_1.reshape(1, H), W_u2a_2, b_u2a_2.reshape(1, H),
      Wih_ue, bih_ue.reshape(1, 3 * H), Whh_ue, bhh_ue.reshape(1, 3 * H),
      Wih_ap, bih_ap.reshape(1, 3 * H), Whh_ap, bhh_ap.reshape(1, 3 * H),
      W_e_1, b_e_1.reshape(1, H), W_e_2, b_e_2.reshape(1, H))
